# baseline (speedup 1.0000x reference)
import functools

import jax
import jax.numpy as jnp
from jax import lax
from jax.experimental import pallas as pl
from jax.experimental.pallas import tpu as pltpu
from jax.experimental.pallas import tpu_sc as plsc

CHUNK = 256
NBUF = 4


@functools.cache
def _build(n_rows_total, dim, chunk, nbuf):
    mesh = plsc.VectorSubcoreMesh(core_axis_name="c", subcore_axis_name="s")
    nc, ns = mesh.num_cores, mesh.num_subcores
    nw = nc * ns
    n_chunks = n_rows_total // (chunk * nw)
    assert n_chunks * chunk * nw == n_rows_total
    n_steps = n_chunks // nbuf
    assert n_steps * nbuf == n_chunks

    @functools.partial(
        pl.kernel,
        out_type=jax.ShapeDtypeStruct((n_rows_total, dim), jnp.float32),
        mesh=mesh,
        compiler_params=pltpu.CompilerParams(use_tc_tiling_on_sc=False),
        scratch_types=[
            pltpu.VMEM((n_chunks, chunk), jnp.int32),
            [pltpu.VMEM((chunk, dim), jnp.float32) for _ in range(nbuf)],
            [pltpu.SemaphoreType.DMA for _ in range(nbuf)],
            [pltpu.SemaphoreType.DMA for _ in range(nbuf)],
        ],
    )
    def gather_kernel(idx_hbm, table_hbm, out_hbm, idx_v, bufs, gsems, wsems):
        wid = lax.axis_index("s") * nc + lax.axis_index("c")
        chunk_base = wid * n_chunks
        pltpu.sync_copy(idx_hbm.at[wid], idx_v)

        def fire_gather(i, b):
            pltpu.async_copy(table_hbm.at[idx_v.at[i]], bufs[b], gsems[b])

        def wait_gather(b):
            pltpu.make_async_copy(
                table_hbm.at[idx_v.at[0]], bufs[b], gsems[b]).wait()

        def fire_write(i, b):
            pltpu.async_copy(
                bufs[b], out_hbm.at[pl.ds((chunk_base + i) * chunk, chunk)],
                wsems[b])

        def wait_write(b):
            pltpu.make_async_copy(
                bufs[b], out_hbm.at[pl.ds(0, chunk)], wsems[b]).wait()

        for b in range(nbuf):
            fire_gather(b, b)

        @pl.loop(0, n_steps - 1)
        def _steady(outer):
            i0 = outer * nbuf
            for b in range(nbuf):
                wait_gather(b)
                fire_write(i0 + b, b)
                wait_write(b)
                fire_gather(i0 + b + nbuf, b)

        i0 = (n_steps - 1) * nbuf
        for b in range(nbuf):
            wait_gather(b)
            fire_write(i0 + b, b)
        for b in range(nbuf):
            wait_write(b)

    return gather_kernel


def kernel(x, weight):
    b0, b1 = x.shape
    dim = weight.shape[1]
    n_rows = b0 * b1
    n_chunks = n_rows // (CHUNK * 32)
    idx = x.reshape(32, n_chunks, CHUNK)
    out = _build(n_rows, dim, CHUNK, NBUF)(idx, weight)
    return out.reshape(b0, b1, dim)


# speedup vs baseline: 1.0007x; 1.0007x over previous
import functools

import jax
import jax.numpy as jnp
from jax import lax
from jax.experimental import pallas as pl
from jax.experimental.pallas import tpu as pltpu
from jax.experimental.pallas import tpu_sc as plsc

NBUF = 8


@functools.cache
def _build(n_b, n_t, dim, nbuf):
    mesh = plsc.VectorSubcoreMesh(core_axis_name="c", subcore_axis_name="s")
    nc, ns = mesh.num_cores, mesh.num_subcores
    nw = nc * ns
    rows_per_w = n_b // nw
    assert rows_per_w * nw == n_b
    n_steps = rows_per_w // nbuf
    assert n_steps * nbuf == rows_per_w

    @functools.partial(
        pl.kernel,
        out_type=jax.ShapeDtypeStruct((n_b, n_t, dim), jnp.float32),
        mesh=mesh,
        compiler_params=pltpu.CompilerParams(use_tc_tiling_on_sc=False),
        scratch_types=[
            pltpu.VMEM((rows_per_w, n_t), jnp.int32),
            [pltpu.VMEM((n_t, dim), jnp.float32) for _ in range(nbuf)],
            [pltpu.SemaphoreType.DMA for _ in range(nbuf)],
            [pltpu.SemaphoreType.DMA for _ in range(nbuf)],
        ],
    )
    def gather_kernel(idx_hbm, table_hbm, out_hbm, idx_v, bufs, gsems, wsems):
        wid = lax.axis_index("s") * nc + lax.axis_index("c")
        row_base = wid * rows_per_w
        pltpu.sync_copy(idx_hbm.at[pl.ds(row_base, rows_per_w)], idx_v)

        def fire_gather(r, b):
            pltpu.async_copy(table_hbm.at[idx_v.at[r]], bufs[b], gsems[b])

        def wait_gather(b):
            pltpu.make_async_copy(
                table_hbm.at[idx_v.at[0]], bufs[b], gsems[b]).wait()

        def fire_write(r, b):
            pltpu.async_copy(bufs[b], out_hbm.at[row_base + r], wsems[b])

        def wait_write(b):
            pltpu.make_async_copy(bufs[b], out_hbm.at[0], wsems[b]).wait()

        for b in range(nbuf):
            fire_gather(b, b)

        @pl.loop(0, n_steps - 1)
        def _steady(outer):
            r0 = outer * nbuf
            for b in range(nbuf):
                wait_gather(b)
                fire_write(r0 + b, b)
                wait_write(b)
                fire_gather(r0 + b + nbuf, b)

        r0 = (n_steps - 1) * nbuf
        for b in range(nbuf):
            wait_gather(b)
            fire_write(r0 + b, b)
        for b in range(nbuf):
            wait_write(b)

    return gather_kernel


def kernel(x, weight):
    n_b, n_t = x.shape
    dim = weight.shape[1]
    return _build(n_b, n_t, dim, NBUF)(x, weight)


# speedup vs baseline: 1.2955x; 1.2947x over previous
import functools

import jax
import jax.numpy as jnp
from jax import lax
from jax.experimental import pallas as pl
from jax.experimental.pallas import tpu as pltpu
from jax.experimental.pallas import tpu_sc as plsc

NBUF = 4


@functools.cache
def _build(n_b, n_t, dpad, nbuf):
    mesh = plsc.VectorSubcoreMesh(core_axis_name="c", subcore_axis_name="s")
    nc, ns = mesh.num_cores, mesh.num_subcores
    nw = nc * ns
    rows_per_w = n_b // nw
    assert rows_per_w * nw == n_b
    n_steps = rows_per_w // nbuf
    assert n_steps * nbuf == rows_per_w

    @functools.partial(
        pl.kernel,
        out_type=jax.ShapeDtypeStruct((n_b, n_t, dpad), jnp.float32),
        mesh=mesh,
        compiler_params=pltpu.CompilerParams(use_tc_tiling_on_sc=True),
        scratch_types=[
            pltpu.VMEM((rows_per_w, n_t), jnp.int32),
            [pltpu.VMEM((n_t, dpad), jnp.float32) for _ in range(nbuf)],
            [pltpu.SemaphoreType.DMA for _ in range(nbuf)],
            [pltpu.SemaphoreType.DMA for _ in range(nbuf)],
        ],
    )
    def gather_kernel(idx_hbm, table_hbm, out_hbm, idx_v, bufs, gsems, wsems):
        wid = lax.axis_index("s") * nc + lax.axis_index("c")
        row_base = wid * rows_per_w
        pltpu.sync_copy(idx_hbm.at[pl.ds(row_base, rows_per_w)], idx_v)

        def fire_gather(r, b):
            pltpu.async_copy(table_hbm.at[idx_v.at[r]], bufs[b], gsems[b])

        def wait_gather(b):
            pltpu.make_async_copy(
                table_hbm.at[idx_v.at[0]], bufs[b], gsems[b]).wait()

        def fire_write(r, b):
            pltpu.async_copy(bufs[b], out_hbm.at[row_base + r], wsems[b])

        def wait_write(b):
            pltpu.make_async_copy(bufs[b], out_hbm.at[0], wsems[b]).wait()

        for b in range(nbuf):
            fire_gather(b, b)

        @pl.loop(0, n_steps - 1)
        def _steady(outer):
            r0 = outer * nbuf
            for b in range(nbuf):
                wait_gather(b)
                fire_write(r0 + b, b)
                wait_write(b)
                fire_gather(r0 + b + nbuf, b)

        r0 = (n_steps - 1) * nbuf
        for b in range(nbuf):
            wait_gather(b)
            fire_write(r0 + b, b)
        for b in range(nbuf):
            wait_write(b)

    return gather_kernel


def kernel(x, weight):
    n_b, n_t = x.shape
    vocab, dim = weight.shape
    dpad = 128
    table = jnp.pad(weight, ((0, 0), (0, dpad - dim)))
    out = _build(n_b, n_t, dpad, NBUF)(x, table)
    return out[:, :, :dim]
